# baseline (speedup 1.0000x reference)
import functools

import jax
import jax.numpy as jnp
from jax import lax
from jax.experimental import pallas as pl
from jax.experimental.pallas import tpu as pltpu
from jax.experimental.pallas import tpu_sc as plsc

S = 14
D = 14
BATCH = 1024
CELLS = BATCH * S * S
WORDS = CELLS * D

NC = 2
NS = 16
NW = NC * NS
LANES = 16

CELLS_PER_W = CELLS // NW
NCHUNK = 4
CHUNK_CELLS = CELLS_PER_W // NCHUNK
CHUNK_WORDS = CHUNK_CELLS * D
GROUPS = CHUNK_CELLS // LANES
WORDS_PER_W = CELLS_PER_W * D


def _sqrt_nr(x):
    i = lax.bitcast_convert_type(x, jnp.int32)
    i = jnp.int32(0x5F3759DF) - lax.shift_right_arithmetic(i, 1)
    y = lax.bitcast_convert_type(i, jnp.float32)
    for _ in range(3):
        y = y * (1.5 - 0.5 * x * y * y)
    return x * y


def _chunk_loss(bp, bt, acc0):
    iota14 = lax.iota(jnp.int32, LANES) * D

    @plsc.parallel_loop(0, GROUPS, carry=acc0, unroll=4)
    def body(g, acc):
        base = g * (LANES * D) + iota14

        def gp(c):
            return plsc.load_gather(bp, [base + c])

        def gt(c):
            return plsc.load_gather(bt, [base + c])

        p0, t0 = gp(0), gt(0)
        p1, t1 = gp(1), gt(1)
        p2, t2 = gp(2), gt(2)
        p3, t3 = gp(3), gt(3)
        p4, t4 = gp(4), gt(4)
        p9, t9 = gp(9), gt(9)

        d0 = p0 - t0
        d1 = p1 - t1
        loc = d0 * d0 + d1 * d1
        loc = loc + (p2 + t2 - 2.0 * _sqrt_nr(p2 * t2))
        loc = loc + (p3 + t3 - 2.0 * _sqrt_nr(p3 * t3))

        cls = jnp.zeros((LANES,), jnp.float32)
        for c in range(10, 14):
            dc = gp(c) - gt(c)
            cls = cls + dc * dc

        d4 = p4 - t4
        d9 = p9 - t9
        zero = jnp.zeros((LANES,), jnp.float32)
        one = jnp.ones((LANES,), jnp.float32)
        coo = jnp.where(t4 > 0.0, one, zero)
        noo = jnp.where(t4 == 0.0, one, zero)

        cell = coo * (5.0 * loc + 2.0 * p4 * p4 + p9 * p9 + cls)
        cell = cell + 0.5 * noo * (d4 * d4 + d9 * d9)
        return acc + cell

    return body


def _make_sc_call():
    mesh = plsc.VectorSubcoreMesh(core_axis_name="c", subcore_axis_name="s")

    @functools.partial(
        pl.kernel,
        mesh=mesh,
        compiler_params=pltpu.CompilerParams(needs_layout_passes=False),
        out_type=jax.ShapeDtypeStruct((NW, LANES), jnp.float32),
        scratch_types=[
            pltpu.VMEM((CHUNK_WORDS,), jnp.float32),
            pltpu.VMEM((CHUNK_WORDS,), jnp.float32),
            pltpu.VMEM((CHUNK_WORDS,), jnp.float32),
            pltpu.VMEM((CHUNK_WORDS,), jnp.float32),
            pltpu.VMEM((LANES,), jnp.float32),
            pltpu.SemaphoreType.DMA,
            pltpu.SemaphoreType.DMA,
            pltpu.SemaphoreType.DMA,
            pltpu.SemaphoreType.DMA,
        ],
    )
    def sc_loss(pred_hbm, targ_hbm, out_hbm,
                bp0, bp1, bt0, bt1, acc_v, sp0, sp1, st0, st1):
        wid = lax.axis_index("s") * NC + lax.axis_index("c")
        wbase = wid * WORDS_PER_W

        bps = (bp0, bp1)
        bts = (bt0, bt1)
        sps = (sp0, sp1)
        sts = (st0, st1)

        def start(k):
            buf = k & 1
            off = wbase + k * CHUNK_WORDS
            hp = pltpu.async_copy(
                pred_hbm.at[pl.ds(off, CHUNK_WORDS)], bps[buf], sps[buf])
            ht = pltpu.async_copy(
                targ_hbm.at[pl.ds(off, CHUNK_WORDS)], bts[buf], sts[buf])
            return hp, ht

        pending = start(0)
        acc = jnp.zeros((LANES,), jnp.float32)
        for k in range(NCHUNK):
            nxt = start(k + 1) if k + 1 < NCHUNK else None
            pending[0].wait()
            pending[1].wait()
            buf = k & 1
            acc = _chunk_loss(bps[buf], bts[buf], acc)
            pending = nxt

        acc_v[...] = acc * (1.0 / BATCH)
        pltpu.sync_copy(acc_v, out_hbm.at[wid])

    return sc_loss


_sc_loss_call = _make_sc_call()


def kernel(pred_tensor, target_tensor):
    p = pred_tensor.reshape(WORDS)
    t = target_tensor.reshape(WORDS)
    partials = _sc_loss_call(p, t)
    return jnp.sum(partials)


# speedup vs baseline: 5.4440x; 5.4440x over previous
import functools

import jax
import jax.numpy as jnp
from jax import lax
from jax.experimental import pallas as pl
from jax.experimental.pallas import tpu as pltpu
from jax.experimental.pallas import tpu_sc as plsc

S = 14
D = 14
BATCH = 1024
PLANES = S * S
NC = 2
NS = 16
NW = NC * NS
LANES = 16

BASE_PLANES = PLANES // NW
EXTRA_FROM = NW - (PLANES - BASE_PLANES * NW)
JVECS = BATCH // LANES


def _sqrt_nr(x):
    i = lax.bitcast_convert_type(x, jnp.int32)
    i = jnp.int32(0x5F3759DF) - lax.shift_right_arithmetic(i, 1)
    y = lax.bitcast_convert_type(i, jnp.float32)
    for _ in range(3):
        y = y * (1.5 - 0.5 * x * y * y)
    return x * y


def _plane_loss(bp, bt, acc0):
    @plsc.parallel_loop(0, JVECS, carry=acc0, unroll=4)
    def body(j, acc):
        b0 = j * LANES

        def gp(c):
            return bp[c, pl.ds(b0, LANES)]

        def gt(c):
            return bt[c, pl.ds(b0, LANES)]

        p0, t0 = gp(0), gt(0)
        p1, t1 = gp(1), gt(1)
        p2, t2 = gp(2), gt(2)
        p3, t3 = gp(3), gt(3)
        p4, t4 = gp(4), gt(4)
        p9, t9 = gp(9), gt(9)

        d0 = p0 - t0
        d1 = p1 - t1
        loc = d0 * d0 + d1 * d1
        loc = loc + (p2 + t2 - 2.0 * _sqrt_nr(p2 * t2))
        loc = loc + (p3 + t3 - 2.0 * _sqrt_nr(p3 * t3))

        cls = jnp.zeros((LANES,), jnp.float32)
        for c in range(10, 14):
            dc = gp(c) - gt(c)
            cls = cls + dc * dc

        d4 = p4 - t4
        d9 = p9 - t9
        zero = jnp.zeros((LANES,), jnp.float32)
        one = jnp.ones((LANES,), jnp.float32)
        coo = jnp.where(t4 > 0.0, one, zero)
        noo = jnp.where(t4 == 0.0, one, zero)

        cell = coo * (5.0 * loc + 2.0 * p4 * p4 + p9 * p9 + cls)
        cell = cell + 0.5 * noo * (d4 * d4 + d9 * d9)
        return acc + cell

    return body


def _make_sc_call():
    mesh = plsc.VectorSubcoreMesh(core_axis_name="c", subcore_axis_name="s")

    @functools.partial(
        pl.kernel,
        mesh=mesh,
        compiler_params=pltpu.CompilerParams(needs_layout_passes=False),
        out_type=jax.ShapeDtypeStruct((NW, LANES), jnp.float32),
        scratch_types=[
            pltpu.VMEM((D, BATCH), jnp.float32),
            pltpu.VMEM((D, BATCH), jnp.float32),
            pltpu.VMEM((D, BATCH), jnp.float32),
            pltpu.VMEM((D, BATCH), jnp.float32),
            pltpu.VMEM((LANES,), jnp.float32),
            pltpu.SemaphoreType.DMA,
            pltpu.SemaphoreType.DMA,
            pltpu.SemaphoreType.DMA,
            pltpu.SemaphoreType.DMA,
        ],
    )
    def sc_loss(pred_hbm, targ_hbm, out_hbm,
                bp0, bp1, bt0, bt1, acc_v, sp0, sp1, st0, st1):
        wid = lax.axis_index("s") * NC + lax.axis_index("c")
        pbase = wid * BASE_PLANES

        bps = (bp0, bp1)
        bts = (bt0, bt1)
        sps = (sp0, sp1)
        sts = (st0, st1)

        def start(plane, s):
            buf = s & 1
            hp = pltpu.async_copy(pred_hbm.at[plane], bps[buf], sps[buf])
            ht = pltpu.async_copy(targ_hbm.at[plane], bts[buf], sts[buf])
            return hp, ht

        pending = start(pbase, 0)
        acc = jnp.zeros((LANES,), jnp.float32)
        for s in range(BASE_PLANES):
            nxt = start(pbase + s + 1, s + 1) if s + 1 < BASE_PLANES else None
            pending[0].wait()
            pending[1].wait()
            acc = _plane_loss(bps[s & 1], bts[s & 1], acc)
            pending = nxt

        extra_acc_v = acc_v
        @pl.when(wid >= EXTRA_FROM)
        def _():
            plane = BASE_PLANES * NW + (wid - EXTRA_FROM)
            h = start(plane, BASE_PLANES)
            h[0].wait()
            h[1].wait()
            extra = _plane_loss(bps[BASE_PLANES & 1], bts[BASE_PLANES & 1],
                                jnp.zeros((LANES,), jnp.float32))
            extra_acc_v[...] = extra

        @pl.when(wid < EXTRA_FROM)
        def _():
            extra_acc_v[...] = jnp.zeros((LANES,), jnp.float32)

        acc_v[...] = (acc + extra_acc_v[...]) * (1.0 / BATCH)
        pltpu.sync_copy(acc_v, out_hbm.at[wid])

    return sc_loss


_sc_loss_call = _make_sc_call()


def kernel(pred_tensor, target_tensor):
    pt = jnp.transpose(pred_tensor, (1, 2, 3, 0)).reshape(PLANES, D, BATCH)
    tt = jnp.transpose(target_tensor, (1, 2, 3, 0)).reshape(PLANES, D, BATCH)
    partials = _sc_loss_call(pt, tt)
    return jnp.sum(partials)


# speedup vs baseline: 5.8707x; 1.0784x over previous
import functools

import jax
import jax.numpy as jnp
from jax import lax
from jax.experimental import pallas as pl
from jax.experimental.pallas import tpu as pltpu
from jax.experimental.pallas import tpu_sc as plsc

S = 14
D = 14
BATCH = 1024
PLANES = S * S
NC = 2
NS = 16
NW = NC * NS
LANES = 16

BASE_PLANES = PLANES // NW
EXTRA_FROM = NW - (PLANES - BASE_PLANES * NW)
JVECS = BATCH // LANES


def _sqrt_nr(x):
    i = lax.bitcast_convert_type(x, jnp.int32)
    i = jnp.int32(0x5F3759DF) - lax.shift_right_arithmetic(i, 1)
    y = lax.bitcast_convert_type(i, jnp.float32)
    for _ in range(3):
        y = y * (1.5 - 0.5 * x * y * y)
    return x * y


def _plane_loss(bp, bt, acc0):
    @plsc.parallel_loop(0, JVECS, carry=acc0, unroll=4)
    def body(j, acc):
        b0 = j * LANES

        def gp(c):
            return bp[c, pl.ds(b0, LANES)]

        def gt(c):
            return bt[c, pl.ds(b0, LANES)]

        p0, t0 = gp(0), gt(0)
        p1, t1 = gp(1), gt(1)
        p2, t2 = gp(2), gt(2)
        p3, t3 = gp(3), gt(3)
        p4, t4 = gp(4), gt(4)
        p9, t9 = gp(9), gt(9)

        d0 = p0 - t0
        d1 = p1 - t1
        loc = d0 * d0 + d1 * d1
        loc = loc + (p2 + t2 - 2.0 * _sqrt_nr(p2 * t2))
        loc = loc + (p3 + t3 - 2.0 * _sqrt_nr(p3 * t3))

        cls = jnp.zeros((LANES,), jnp.float32)
        for c in range(10, 14):
            dc = gp(c) - gt(c)
            cls = cls + dc * dc

        d4 = p4 - t4
        d9 = p9 - t9
        zero = jnp.zeros((LANES,), jnp.float32)
        one = jnp.ones((LANES,), jnp.float32)
        coo = jnp.where(t4 > 0.0, one, zero)
        noo = jnp.where(t4 == 0.0, one, zero)

        cell = coo * (5.0 * loc + 2.0 * p4 * p4 + p9 * p9 + cls)
        cell = cell + 0.5 * noo * (d4 * d4 + d9 * d9)
        return acc + cell

    return body


def _make_sc_call():
    mesh = plsc.VectorSubcoreMesh(core_axis_name="c", subcore_axis_name="s")

    @functools.partial(
        pl.kernel,
        mesh=mesh,
        compiler_params=pltpu.CompilerParams(needs_layout_passes=False),
        out_type=jax.ShapeDtypeStruct((NW, LANES), jnp.float32),
        scratch_types=[
            pltpu.VMEM((D, BATCH), jnp.float32),
            pltpu.VMEM((D, BATCH), jnp.float32),
            pltpu.VMEM((D, BATCH), jnp.float32),
            pltpu.VMEM((D, BATCH), jnp.float32),
            pltpu.VMEM((LANES,), jnp.float32),
            pltpu.SemaphoreType.DMA,
            pltpu.SemaphoreType.DMA,
            pltpu.SemaphoreType.DMA,
            pltpu.SemaphoreType.DMA,
        ],
    )
    def sc_loss(pred_hbm, targ_hbm, out_hbm,
                bp0, bp1, bt0, bt1, acc_v, sp0, sp1, st0, st1):
        wid = lax.axis_index("s") * NC + lax.axis_index("c")
        pbase = wid * BASE_PLANES

        bps = (bp0, bp1)
        bts = (bt0, bt1)
        sps = (sp0, sp1)
        sts = (st0, st1)

        def start(plane, buf):
            pltpu.async_copy(pred_hbm.at[plane], bps[buf], sps[buf])
            pltpu.async_copy(targ_hbm.at[plane], bts[buf], sts[buf])

        def wait(buf):
            pltpu.make_async_copy(pred_hbm.at[0], bps[buf], sps[buf]).wait()
            pltpu.make_async_copy(targ_hbm.at[0], bts[buf], sts[buf]).wait()

        start(pbase, 0)
        start(pbase + 1, 1)

        npairs = BASE_PLANES // 2

        def body(s, acc):
            wait(0)
            acc = _plane_loss(bp0, bt0, acc)

            @pl.when(s < npairs - 1)
            def _():
                start(pbase + 2 * s + 2, 0)

            wait(1)
            acc = _plane_loss(bp1, bt1, acc)

            @pl.when(s < npairs - 1)
            def _():
                start(pbase + 2 * s + 3, 1)

            return acc

        acc = lax.fori_loop(0, npairs, body, jnp.zeros((LANES,), jnp.float32))

        @pl.when(wid >= EXTRA_FROM)
        def _():
            start(BASE_PLANES * NW + (wid - EXTRA_FROM), 0)
            wait(0)
            acc_v[...] = _plane_loss(bp0, bt0, jnp.zeros((LANES,), jnp.float32))

        @pl.when(wid < EXTRA_FROM)
        def _():
            acc_v[...] = jnp.zeros((LANES,), jnp.float32)

        acc_v[...] = (acc + acc_v[...]) * (1.0 / BATCH)
        pltpu.sync_copy(acc_v, out_hbm.at[wid])

    return sc_loss


_sc_loss_call = _make_sc_call()


def kernel(pred_tensor, target_tensor):
    pt = jnp.transpose(pred_tensor, (1, 2, 3, 0)).reshape(PLANES, D, BATCH)
    tt = jnp.transpose(target_tensor, (1, 2, 3, 0)).reshape(PLANES, D, BATCH)
    partials = _sc_loss_call(pt, tt)
    return jnp.sum(partials)
